# Initial kernel scaffold; baseline (speedup 1.0000x reference)
import jax
import jax.numpy as jnp
from jax.experimental import pallas as pl


def kernel(h, edge_index, W1, a1, W2, a2, Wres2):
    raise NotImplementedError("write your pallas kernel here")


# speedup vs baseline: 9.5783x; 9.5783x over previous
import jax
import jax.numpy as jnp
from jax import lax
from jax.experimental import pallas as pl
from jax.experimental.pallas import tpu as pltpu
from jax.experimental.pallas import tpu_sc as plsc

N = 10000
NA = 10240
E = 160000
NEG_SLOPE = 0.2

NC = 2
NS = 16
CH = 64
G = CH // 16
TE = 1000
TEB = -(-TE // CH)
RT = NA // NS
ZR = 32
XR = NA // 128


def _mm_body(x_ref, w_ref, o_ref):
    o_ref[...] = jnp.dot(x_ref[...], w_ref[...],
                         preferred_element_type=jnp.float32)


def _matmul(x, w, bm):
    m, k = x.shape
    _, n = w.shape
    return pl.pallas_call(
        _mm_body,
        grid=(m // bm,),
        in_specs=[
            pl.BlockSpec((bm, k), lambda i: (i, 0)),
            pl.BlockSpec((k, n), lambda i: (0, 0)),
        ],
        out_specs=pl.BlockSpec((bm, n), lambda i: (i, 0)),
        out_shape=jax.ShapeDtypeStruct((m, n), jnp.float32),
    )(x, w)


def _make_agg(d, gw, idx_stride, n_passes, EC, tail, adim):
    split_ex = n_passes > 1
    nbl = EC // TE

    def body(*refs):
        if split_ex:
            (featv, ids, avec, out, out_ex,
             a_v, id_st, sidx, didx, acc, acc_ex,
             srows, drows, obuf, exbuf, exrow, zbuf, sem_s, sem_d) = refs
        else:
            (featv, ids, avec, out,
             a_v, id_st, sidx, didx, acc,
             srows, drows, obuf, zbuf, sem_s, sem_d) = refs
        c = lax.axis_index("c")
        s = lax.axis_index("s")
        iota = lax.broadcasted_iota(jnp.int32, (16,), 0)
        zi = jnp.zeros((16,), jnp.int32)
        zv = jnp.zeros((16,), jnp.float32)

        pltpu.sync_copy(avec, a_v)

        def zrow(i, _):
            for u in range(8):
                zbuf[i, pl.ds(u * 16, 16)] = zv
            return 0
        lax.fori_loop(0, ZR, zrow, 0)
        if split_ex:
            def zxrow(i, _):
                for u in range(8):
                    exrow[i, pl.ds(u * 16, 16)] = zv
                return 0
            lax.fori_loop(0, 16, zxrow, 0)
        else:
            def zorow(i, _):
                for u in range(8):
                    obuf[i, pl.ds(u * 16, 16)] = zv
                return 0
            lax.fori_loop(0, CH, zorow, 0)

        if split_ex:
            eb = s * EC
        else:
            eb = (c * NS + s) * EC

        for p in range(n_passes):
            if split_ex:
                head = c * n_passes + p
                plane = head
            else:
                head = 0
                plane = c

            for q in range(RT // ZR):
                pltpu.sync_copy(zbuf, acc.at[pl.ds(s * RT + q * ZR, ZR)])
            if split_ex:
                @pl.when(s == 0)
                def _zx():
                    pltpu.sync_copy(zbuf.at[pl.ds(0, XR)], acc_ex)
            plsc.subcore_barrier()

            def block(t, _):
              pltpu.sync_copy(ids.at[pl.ds(eb + t * TE, TE)], id_st)

              def chunk(ch, _):
                base = ch * CH
                for g in range(G):
                    off = base + g * 16
                    cb = id_st[pl.ds(off, 16)]
                    sv = cb & 16383
                    dv = cb >> 14
                    valid = (off + iota) < TE
                    sv = jnp.where(valid, sv, 0)
                    dv = jnp.where(valid, dv, 0)
                    if idx_stride > 1:
                        sv = sv * idx_stride + head
                        dv = dv * idx_stride + head
                    sidx[pl.ds(g * 16, 16)] = sv
                    didx[pl.ds(g * 16, 16)] = dv
                cp1 = pltpu.async_copy(featv.at[sidx], srows, sem_s)
                cp2 = pltpu.async_copy(featv.at[didx], drows, sem_d)
                cp1.wait()
                cp2.wait()

                def edge(e, _):
                    accv = None
                    svs = []
                    for u in range(d // 16):
                        sl = pl.ds(u * 16, 16)
                        svv = srows[e, sl]
                        dvv = drows[e, sl]
                        svs.append(svv)
                        t2 = svv + dvv
                        lr = jnp.maximum(t2, NEG_SLOPE * t2)
                        if split_ex:
                            pterm = lr * a_v[pl.ds(head * d + u * 16, 16)]
                        else:
                            pterm = lr * a_v[pl.ds(u * 16, 16)]
                        accv = pterm if accv is None else accv + pterm
                    ex = jnp.exp(jnp.broadcast_to(jnp.sum(accv), (16,)))
                    for u in range(d // 16):
                        obuf[e, pl.ds(u * 16, 16)] = svs[u] * ex
                    if split_ex:
                        exbuf[e, pl.ds(0, 16)] = ex
                    else:
                        obuf[e, pl.ds(d, 16)] = jnp.where(iota == 0, ex, 0.0)
                    return 0
                lax.fori_loop(0, CH, edge, 0)

                for g in range(G):
                    off = base + g * 16
                    dv = id_st[pl.ds(off, 16)] >> 14
                    valid = (off + iota) < TE
                    dv = jnp.where(valid, dv, N + iota)
                    pltpu.sync_copy(obuf.at[pl.ds(g * 16, 16)],
                                    acc.at[dv], add=True)
                    if split_ex:
                        ex16 = plsc.load_gather(exbuf, [g * 16 + iota, zi])
                        lo = dv & 127
                        plsc.store_scatter(exrow, [iota, lo], ex16)
                        pltpu.sync_copy(exrow, acc_ex.at[dv >> 7], add=True)
                        plsc.store_scatter(exrow, [iota, lo], zv)
                return 0
              lax.fori_loop(0, TEB, chunk, 0)
              return 0
            lax.fori_loop(0, nbl, block, 0)

            plsc.subcore_barrier()
            for q in range(RT // ZR):
                r0 = s * RT + q * ZR
                pltpu.sync_copy(acc.at[pl.ds(r0, ZR)],
                                out.at[plane, pl.ds(r0, ZR)])
            if split_ex:
                @pl.when(s == 0)
                def _cx():
                    pltpu.sync_copy(acc_ex, out_ex.at[plane])

    mesh = plsc.VectorSubcoreMesh(core_axis_name="c", subcore_axis_name="s")
    planes = NC * n_passes
    out_type = jax.ShapeDtypeStruct((planes, NA, 128), jnp.float32)
    if split_ex:
        out_type = (out_type,
                    jax.ShapeDtypeStruct((planes, XR, 128), jnp.float32))
    scratch = [
        pltpu.VMEM((adim,), jnp.float32),
        pltpu.VMEM((TE,), jnp.int32),
        pltpu.VMEM((CH,), jnp.int32),
        pltpu.VMEM((CH,), jnp.int32),
        pltpu.VMEM_SHARED((NA, 128), jnp.float32),
    ]
    if split_ex:
        scratch.append(pltpu.VMEM_SHARED((XR, 128), jnp.float32))
    scratch += [
        pltpu.VMEM((CH, gw), jnp.float32),
        pltpu.VMEM((CH, gw), jnp.float32),
        pltpu.VMEM((CH, 128), jnp.float32),
    ]
    if split_ex:
        scratch += [
            pltpu.VMEM((CH, 16), jnp.float32),
            pltpu.VMEM((16, 128), jnp.float32),
        ]
    scratch += [
        pltpu.VMEM((ZR, 128), jnp.float32),
        pltpu.SemaphoreType.DMA,
        pltpu.SemaphoreType.DMA,
    ]
    return pl.kernel(
        body,
        out_type=out_type,
        mesh=mesh,
        scratch_types=scratch,
        compiler_params=pltpu.CompilerParams(needs_layout_passes=False),
    )


_agg1 = _make_agg(d=128, gw=128, idx_stride=4, n_passes=2, EC=E // NS,
                  tail=True, adim=512)
_agg2 = _make_agg(d=64, gw=128, idx_stride=1, n_passes=1, EC=E // (NC * NS),
                  tail=True, adim=64)


def _fin1_body(ad_ref, dn_ref, w_ref, o_ref):
    o = None
    for k in range(4):
        x = ad_ref[k, :, :] / (dn_ref[k, :, :] + 1e-9)
        x = jnp.where(x > 0.0, x, jnp.exp(jnp.minimum(x, 0.0)) - 1.0)
        pk = jnp.dot(x, w_ref[k * 128:(k + 1) * 128, :],
                     preferred_element_type=jnp.float32)
        o = pk if o is None else o + pk
    o_ref[...] = o


def _fin1(ad, dn, wcat, bm=400):
    return pl.pallas_call(
        _fin1_body,
        grid=(N // bm,),
        in_specs=[
            pl.BlockSpec((4, bm, 128), lambda i: (0, i, 0)),
            pl.BlockSpec((4, bm, 1), lambda i: (0, i, 0)),
            pl.BlockSpec((512, 128), lambda i: (0, 0)),
        ],
        out_specs=pl.BlockSpec((bm, 128), lambda i: (i, 0)),
        out_shape=jax.ShapeDtypeStruct((N, 128), jnp.float32),
    )(ad, dn, wcat)


def _fin2_body(ad_ref, f2_ref, o_ref):
    num = ad_ref[0, :, 0:64] + ad_ref[1, :, 0:64]
    dn = ad_ref[0, :, 64:65] + ad_ref[1, :, 64:65] + 1e-9
    o_ref[...] = num / dn + f2_ref[:, 64:128]


def _fin2(ad, res, bm=400):
    return pl.pallas_call(
        _fin2_body,
        grid=(N // bm,),
        in_specs=[
            pl.BlockSpec((2, bm, 128), lambda i: (0, i, 0)),
            pl.BlockSpec((bm, 128), lambda i: (i, 0)),
        ],
        out_specs=pl.BlockSpec((bm, 64), lambda i: (i, 0)),
        out_shape=jax.ShapeDtypeStruct((N, 64), jnp.float32),
    )(ad, res)


def kernel(h, edge_index, W1, a1, W2, a2, Wres2):
    cmb = edge_index[1] * 16384 + edge_index[0]
    feat1 = _matmul(h, W1, 400)
    featv1 = feat1.reshape(N * 4, 128)
    ad1, ex1 = _agg1(featv1, cmb, a1.reshape(-1))
    dn1 = ex1.reshape(4, NA, 1)
    wcat = jnp.concatenate([W2, Wres2], axis=1)
    f2 = _fin1(ad1, dn1, wcat)
    ad2 = _agg2(f2, cmb, a2.reshape(-1))
    return _fin2(ad2, f2)


# speedup vs baseline: 9.8362x; 1.0269x over previous
import jax
import jax.numpy as jnp
from jax import lax
from jax.experimental import pallas as pl
from jax.experimental.pallas import tpu as pltpu
from jax.experimental.pallas import tpu_sc as plsc

N = 10000
NA = 10240
E = 160000
NEG_SLOPE = 0.2

NC = 2
NS = 16
CH = 64
G = CH // 16
TE = 1000
TEB = -(-TE // CH)
RT = NA // NS
ZR = 16
XR = NA // 128


def _mm_body(x_ref, w_ref, o_ref):
    o_ref[...] = jnp.dot(x_ref[...], w_ref[...],
                         preferred_element_type=jnp.float32)


def _matmul(x, w, bm):
    m, k = x.shape
    _, n = w.shape
    return pl.pallas_call(
        _mm_body,
        grid=(m // bm,),
        in_specs=[
            pl.BlockSpec((bm, k), lambda i: (i, 0)),
            pl.BlockSpec((k, n), lambda i: (0, 0)),
        ],
        out_specs=pl.BlockSpec((bm, n), lambda i: (i, 0)),
        out_shape=jax.ShapeDtypeStruct((m, n), jnp.float32),
    )(x, w)


def _make_agg(d, gw, idx_stride, n_passes, EC, tail, adim):
    split_ex = n_passes > 1
    nbl = EC // TE

    def body(*refs):
        if split_ex:
            (featv, ids, avec, out, out_ex,
             a_v, id_st, sidx, didx, dsca, acc, acc_ex, dex,
             srows, drows, obuf, exbuf, exrow, zbuf, sem_s, sem_d) = refs
        else:
            (featv, ids, avec, out,
             a_v, id_st, sidx, didx, dsca, acc,
             srows, drows, obuf, zbuf, sem_s, sem_d) = refs
        c = lax.axis_index("c")
        s = lax.axis_index("s")
        iota = lax.broadcasted_iota(jnp.int32, (16,), 0)
        zi = jnp.zeros((16,), jnp.int32)
        zv = jnp.zeros((16,), jnp.float32)

        pltpu.sync_copy(avec, a_v)

        def zrow(i, _):
            for u in range(8):
                zbuf[i, pl.ds(u * 16, 16)] = zv
            return 0
        lax.fori_loop(0, ZR, zrow, 0)
        if split_ex:
            def zxrow(i, _):
                for u in range(8):
                    exrow[i, pl.ds(u * 16, 16)] = zv
                return 0
            lax.fori_loop(0, CH, zxrow, 0)
        else:
            def zorow(i, _):
                for u in range(8):
                    obuf[i, pl.ds(u * 16, 16)] = zv
                return 0
            lax.fori_loop(0, CH, zorow, 0)

        if split_ex:
            eb = s * EC
        else:
            eb = (c * NS + s) * EC

        for p in range(n_passes):
            if split_ex:
                head = c * n_passes + p
                plane = head
            else:
                head = 0
                plane = c

            for q in range(RT // ZR):
                pltpu.sync_copy(zbuf, acc.at[pl.ds(s * RT + q * ZR, ZR)])
            if split_ex:
                @pl.when(s == 0)
                def _zx():
                    for q in range(XR // ZR):
                        pltpu.sync_copy(zbuf, acc_ex.at[pl.ds(q * ZR, ZR)])
            plsc.subcore_barrier()

            def block(t, _):
              pltpu.sync_copy(ids.at[pl.ds(eb + t * TE, TE)], id_st)

              def chunk(ch, _):
                base = ch * CH
                for g in range(G):
                    off = base + g * 16
                    cb = id_st[pl.ds(off, 16)]
                    sv = cb & 16383
                    dv = cb >> 14
                    valid = (off + iota) < TE
                    sv = jnp.where(valid, sv, 0)
                    dv = jnp.where(valid, dv, 0)
                    if idx_stride > 1:
                        sv = sv * idx_stride + head
                        dv = dv * idx_stride + head
                    sidx[pl.ds(g * 16, 16)] = sv
                    didx[pl.ds(g * 16, 16)] = dv
                cp1 = pltpu.async_copy(featv.at[sidx], srows, sem_s)
                cp2 = pltpu.async_copy(featv.at[didx], drows, sem_d)
                cp1.wait()
                cp2.wait()

                def edge(e, _):
                    accv = None
                    svs = []
                    for u in range(d // 16):
                        sl = pl.ds(u * 16, 16)
                        svv = srows[e, sl]
                        dvv = drows[e, sl]
                        svs.append(svv)
                        t2 = svv + dvv
                        lr = jnp.maximum(t2, NEG_SLOPE * t2)
                        if split_ex:
                            pterm = lr * a_v[pl.ds(head * d + u * 16, 16)]
                        else:
                            pterm = lr * a_v[pl.ds(u * 16, 16)]
                        accv = pterm if accv is None else accv + pterm
                    ex = jnp.exp(jnp.broadcast_to(jnp.sum(accv), (16,)))
                    for u in range(d // 16):
                        obuf[e, pl.ds(u * 16, 16)] = svs[u] * ex
                    if split_ex:
                        exbuf[e, pl.ds(0, 16)] = ex
                    else:
                        obuf[e, pl.ds(d, 16)] = jnp.where(iota == 0, ex, 0.0)
                    return 0
                lax.fori_loop(0, CH, edge, 0)

                for g in range(G):
                    off = base + g * 16
                    dv = id_st[pl.ds(off, 16)] >> 14
                    valid = (off + iota) < TE
                    dvm = jnp.where(valid, dv, N + iota)
                    dsca[pl.ds(g * 16, 16)] = dvm
                    if split_ex:
                        ex16 = plsc.load_gather(exbuf, [g * 16 + iota, zi])
                        plsc.store_scatter(exrow,
                                           [g * 16 + iota, dvm & 127], ex16)
                        dex[pl.ds(g * 16, 16)] = dvm >> 7
                pltpu.sync_copy(obuf, acc.at[dsca], add=True)
                if split_ex:
                    pltpu.sync_copy(exrow, acc_ex.at[dex], add=True)
                    for g in range(G):
                        lo = dsca[pl.ds(g * 16, 16)] & 127
                        plsc.store_scatter(exrow, [g * 16 + iota, lo], zv)
                return 0
              lax.fori_loop(0, TEB, chunk, 0)
              return 0
            lax.fori_loop(0, nbl, block, 0)

            plsc.subcore_barrier()
            for q in range(RT // ZR):
                r0 = s * RT + q * ZR
                pltpu.sync_copy(acc.at[pl.ds(r0, ZR)],
                                out.at[plane, pl.ds(r0, ZR)])
            if split_ex:
                @pl.when(s == 0)
                def _cx():
                    pltpu.sync_copy(acc_ex, out_ex.at[plane])

    mesh = plsc.VectorSubcoreMesh(core_axis_name="c", subcore_axis_name="s")
    planes = NC * n_passes
    out_type = jax.ShapeDtypeStruct((planes, NA, 128), jnp.float32)
    if split_ex:
        out_type = (out_type,
                    jax.ShapeDtypeStruct((planes, XR, 128), jnp.float32))
    scratch = [
        pltpu.VMEM((adim,), jnp.float32),
        pltpu.VMEM((TE,), jnp.int32),
        pltpu.VMEM((CH,), jnp.int32),
        pltpu.VMEM((CH,), jnp.int32),
        pltpu.VMEM((CH,), jnp.int32),
        pltpu.VMEM_SHARED((NA, 128), jnp.float32),
    ]
    if split_ex:
        scratch.append(pltpu.VMEM_SHARED((XR, 128), jnp.float32))
        scratch.append(pltpu.VMEM((CH,), jnp.int32))
    scratch += [
        pltpu.VMEM((CH, gw), jnp.float32),
        pltpu.VMEM((CH, gw), jnp.float32),
        pltpu.VMEM((CH, 128), jnp.float32),
    ]
    if split_ex:
        scratch += [
            pltpu.VMEM((CH, 16), jnp.float32),
            pltpu.VMEM((CH, 128), jnp.float32),
        ]
    scratch += [
        pltpu.VMEM((ZR, 128), jnp.float32),
        pltpu.SemaphoreType.DMA,
        pltpu.SemaphoreType.DMA,
    ]
    return pl.kernel(
        body,
        out_type=out_type,
        mesh=mesh,
        scratch_types=scratch,
        compiler_params=pltpu.CompilerParams(needs_layout_passes=False),
    )


_agg1 = _make_agg(d=128, gw=128, idx_stride=4, n_passes=2, EC=E // NS,
                  tail=True, adim=512)
_agg2 = _make_agg(d=64, gw=128, idx_stride=1, n_passes=1, EC=E // (NC * NS),
                  tail=True, adim=64)


def _fin1_body(ad_ref, dn_ref, w_ref, o_ref):
    o = None
    for k in range(4):
        x = ad_ref[k, :, :] / (dn_ref[k, :, :] + 1e-9)
        x = jnp.where(x > 0.0, x, jnp.exp(jnp.minimum(x, 0.0)) - 1.0)
        pk = jnp.dot(x, w_ref[k * 128:(k + 1) * 128, :],
                     preferred_element_type=jnp.float32)
        o = pk if o is None else o + pk
    o_ref[...] = o


def _fin1(ad, dn, wcat, bm=400):
    return pl.pallas_call(
        _fin1_body,
        grid=(N // bm,),
        in_specs=[
            pl.BlockSpec((4, bm, 128), lambda i: (0, i, 0)),
            pl.BlockSpec((4, bm, 1), lambda i: (0, i, 0)),
            pl.BlockSpec((512, 128), lambda i: (0, 0)),
        ],
        out_specs=pl.BlockSpec((bm, 128), lambda i: (i, 0)),
        out_shape=jax.ShapeDtypeStruct((N, 128), jnp.float32),
    )(ad, dn, wcat)


def _fin2_body(ad_ref, f2_ref, o_ref):
    num = ad_ref[0, :, 0:64] + ad_ref[1, :, 0:64]
    dn = ad_ref[0, :, 64:65] + ad_ref[1, :, 64:65] + 1e-9
    o_ref[...] = num / dn + f2_ref[:, 64:128]


def _fin2(ad, res, bm=400):
    return pl.pallas_call(
        _fin2_body,
        grid=(N // bm,),
        in_specs=[
            pl.BlockSpec((2, bm, 128), lambda i: (0, i, 0)),
            pl.BlockSpec((bm, 128), lambda i: (i, 0)),
        ],
        out_specs=pl.BlockSpec((bm, 64), lambda i: (i, 0)),
        out_shape=jax.ShapeDtypeStruct((N, 64), jnp.float32),
    )(ad, res)


def kernel(h, edge_index, W1, a1, W2, a2, Wres2):
    cmb = edge_index[1] * 16384 + edge_index[0]
    feat1 = _matmul(h, W1, 400)
    featv1 = feat1.reshape(N * 4, 128)
    ad1, ex1 = _agg1(featv1, cmb, a1.reshape(-1))
    dn1 = ex1.reshape(4, NA, 1)
    wcat = jnp.concatenate([W2, Wres2], axis=1)
    f2 = _fin1(ad1, dn1, wcat)
    ad2 = _agg2(f2, cmb, a2.reshape(-1))
    return _fin2(ad2, f2)
